# baseline (speedup 1.0000x reference)
import functools

import jax
import jax.numpy as jnp
from jax import lax
from jax.experimental import pallas as pl
from jax.experimental.pallas import tpu as pltpu
from jax.experimental.pallas import tpu_sc as plsc

F32 = jnp.float32
BF16 = jnp.bfloat16
I32 = jnp.int32
EPS = 1e-8

NC, NS, L = 2, 16, 16
NW = NC * NS


def _pad1(a, n, v):
    return jnp.concatenate([a, jnp.full((n - a.shape[0],), v, a.dtype)])


def _make_prep(N, M, D, R):
    NT = N + M
    nb = NT // R
    nbx = N // R

    def body(x_ref, hef_ref, Wlin_ref, blin_ref, A_ref, B_ref, We1r_ref,
             be1_ref, br_ref, h_ref, hA_ref, hB_ref):
        pid = pl.program_id(0)
        is_node = pid < nbx
        xb = x_ref[:]
        hb = hef_ref[:]
        xw = jnp.dot(xb, Wlin_ref[:], preferred_element_type=F32) + blin_ref[:]
        h = jnp.where(is_node, xw, hb)
        h_ref[:] = h
        bias_e = be1_ref[:] + jnp.dot(br_ref[:], We1r_ref[:],
                                      preferred_element_type=F32)
        hA_ref[:] = jnp.dot(h, A_ref[:], preferred_element_type=F32) + bias_e
        hB_ref[:] = jnp.dot(h, B_ref[:], preferred_element_type=F32)

    row_spec = pl.BlockSpec((R, D), lambda i: (i, 0))
    x_spec = pl.BlockSpec((R, D), lambda i: (jnp.minimum(i, nbx - 1), 0))
    hef_spec = pl.BlockSpec((R, D), lambda i: (jnp.maximum(i - nbx, 0), 0))
    w_spec = pl.BlockSpec((D, D), lambda i: (0, 0))
    v_spec = pl.BlockSpec((1, D), lambda i: (0, 0))
    return pl.pallas_call(
        body,
        grid=(nb,),
        in_specs=[x_spec, hef_spec, w_spec, v_spec, w_spec, w_spec, w_spec,
                  v_spec, v_spec],
        out_specs=[row_spec, row_spec, row_spec],
        out_shape=[jax.ShapeDtypeStruct((NT, D), F32)] * 3,
    )


def _make_mlp(TE, D, BE):
    nb = TE // BE

    def body(pre_ref, rad_ref, Wr_ref, We1r_ref, We2_ref, be2_ref, Wc1_ref,
             bc1_ref, Wc2t_ref, gam_ref, cen_ref, eflo_ref, efhi_ref, s_ref):
        r = rad_ref[:]
        g = gam_ref[0, 0]
        c0 = cen_ref[0, 0]
        c1 = cen_ref[0, 1]
        Wrc = jnp.dot(Wr_ref[:], We1r_ref[:], preferred_element_type=F32)
        rbf0 = jnp.exp(-g * (r - c0) ** 2)
        rbf1 = jnp.exp(-g * (r - c1) ** 2)
        term = rbf0 * Wrc[0:1, :] + rbf1 * Wrc[1:2, :]
        e = pre_ref[:] + term
        e = e * jax.nn.sigmoid(e)
        ef = jnp.dot(e, We2_ref[:], preferred_element_type=F32) + be2_ref[:]
        ef = ef * jax.nn.sigmoid(ef)
        cm = jnp.dot(ef, Wc1_ref[:], preferred_element_type=F32) + bc1_ref[:]
        cm = cm * jax.nn.sigmoid(cm)
        cmt = jnp.tanh(jnp.sum(cm * Wc2t_ref[:], axis=1, keepdims=True))
        s_ref[:] = cmt / (jnp.sqrt(r) + EPS)
        eflo_ref[:] = ef[:, : D // 2]
        efhi_ref[:] = ef[:, D // 2:]

    row_spec = pl.BlockSpec((BE, D), lambda i: (i, 0))
    one_spec = pl.BlockSpec((BE, 1), lambda i: (i, 0))
    w_spec = pl.BlockSpec((D, D), lambda i: (0, 0))
    v_spec = pl.BlockSpec((1, D), lambda i: (0, 0))
    return pl.pallas_call(
        body,
        grid=(nb,),
        in_specs=[pl.BlockSpec((BE, D), lambda i: (i, 0)), one_spec,
                  pl.BlockSpec((2, D), lambda i: (0, 0)),
                  w_spec, w_spec, v_spec, w_spec, v_spec, v_spec,
                  pl.BlockSpec((1, 1), lambda i: (0, 0)),
                  pl.BlockSpec((1, 2), lambda i: (0, 0))],
        out_specs=[pl.BlockSpec((BE, D // 2), lambda i: (i, 0)),
                   pl.BlockSpec((BE, D // 2), lambda i: (i, 0)),
                   one_spec],
        out_shape=[jax.ShapeDtypeStruct((TE, D // 2), F32),
                   jax.ShapeDtypeStruct((TE, D // 2), F32),
                   jax.ShapeDtypeStruct((TE, 1), F32)],
    )


def _make_aux(NT, R):
    nb = NT // R

    def body(a0_ref, a1_ref, mean3_ref, cnt_ref):
        acc = a0_ref[:] + a1_ref[:]
        cnt = jnp.maximum(acc[:, 3:4], 1.0)
        mean3_ref[:] = acc[:, 0:3] / cnt
        cnt_ref[:] = cnt

    return pl.pallas_call(
        body,
        grid=(nb,),
        in_specs=[pl.BlockSpec((R, 16), lambda i: (i, 0))] * 2,
        out_specs=[pl.BlockSpec((R, 3), lambda i: (i, 0)),
                   pl.BlockSpec((R, 1), lambda i: (i, 0))],
        out_shape=[jax.ShapeDtypeStruct((NT, 3), F32),
                   jax.ShapeDtypeStruct((NT, 1), F32)],
    )


def _make_node(NT, D, R):
    nb = NT // R

    def body(h_ref, alo0_ref, alo1_ref, ahi0_ref, ahi1_ref,
             t0_ref, t1_ref, t2_ref, t3_ref, cnt_ref, base_ref,
             Wn1h_ref, Wn1a_ref, bn1_ref, Wn2_ref, bn2_ref,
             hn_ref, coord_ref):
        h = h_ref[:]
        agg = jnp.concatenate([alo0_ref[:] + alo1_ref[:],
                               ahi0_ref[:] + ahi1_ref[:]], axis=1)
        mid = (jnp.dot(h, Wn1h_ref[:], preferred_element_type=F32)
               + jnp.dot(agg, Wn1a_ref[:], preferred_element_type=F32)
               + bn1_ref[:])
        mid = mid * jax.nn.sigmoid(mid)
        out = jnp.dot(mid, Wn2_ref[:], preferred_element_type=F32) + bn2_ref[:]
        hn_ref[:] = h + out
        tsum = (t0_ref[:] + t1_ref[:]) + (t2_ref[:] + t3_ref[:])
        coord_ref[:] = base_ref[:] + tsum[:, 0:3] / cnt_ref[:]

    row_spec = pl.BlockSpec((R, D), lambda i: (i, 0))
    w_spec = pl.BlockSpec((D, D), lambda i: (0, 0))
    v_spec = pl.BlockSpec((1, D), lambda i: (0, 0))
    return pl.pallas_call(
        body,
        grid=(nb,),
        in_specs=[row_spec]
                 + [pl.BlockSpec((R, D // 2), lambda i: (i, 0))] * 4
                 + [pl.BlockSpec((R, 16), lambda i: (i, 0))] * 4
                 + [pl.BlockSpec((R, 1), lambda i: (i, 0)),
                    pl.BlockSpec((R, 3), lambda i: (i, 0)),
                    w_spec, w_spec, v_spec, w_spec, v_spec],
        out_specs=[row_spec, pl.BlockSpec((R, 3), lambda i: (i, 0))],
        out_shape=[jax.ShapeDtypeStruct((NT, D), F32),
                   jax.ShapeDtypeStruct((NT, 3), F32)],
    )


def _make_stage0(N, NT, ACC_R, E_pad):
    per_tile = E_pad // NW
    C = 128
    n_chunks = per_tile // C
    rows_pt = ACC_R // NS
    mesh = plsc.VectorSubcoreMesh(core_axis_name="c", subcore_axis_name="s")

    @functools.partial(
        pl.kernel,
        out_type=[jax.ShapeDtypeStruct((NC, ACC_R, 16), F32),
                  jax.ShapeDtypeStruct((E_pad,), F32)],
        mesh=mesh,
        compiler_params=pltpu.CompilerParams(needs_layout_passes=False, use_tc_tiling_on_sc=False),
        scratch_types=[
            pltpu.VMEM((N,), F32), pltpu.VMEM((N,), F32), pltpu.VMEM((N,), F32),
            pltpu.VMEM((n_chunks, C), I32), pltpu.VMEM((n_chunks, C), I32),
            pltpu.VMEM((n_chunks, C), I32),
            pltpu.VMEM((C, 16), F32), pltpu.VMEM((C, 16), F32),
            pltpu.VMEM((C,), F32),
            pltpu.VMEM((rows_pt // 8, 16), F32),
            pltpu.VMEM_SHARED((ACC_R, 16), F32),
        ],
    )
    def k(ncx_h, ncy_h, ncz_h, row0g2_h, row0c2_h, colc2_h,
          acc_out_h, radial_h,
          cx_v, cy_v, cz_v, ig_all, ic_all, icc_all, crow_v, cnt_v, rho_v,
          zrow_v, acc_s):
        core = lax.axis_index("c")
        sub = lax.axis_index("s")
        wid = core * NS + sub
        lanes = lax.iota(I32, L)
        one_l3 = jnp.where(lanes == 3, 1.0, 0.0).astype(F32)
        zero_l = jnp.zeros((L,), F32)

        def zfill(i, _):
            zrow_v[i] = zero_l
            return 0
        lax.fori_loop(0, rows_pt // 8, zfill, 0)

        def cfill(i, _):
            crow_v[i] = one_l3
            cnt_v[i] = one_l3
            return 0
        lax.fori_loop(0, C, cfill, 0)

        for j in range(8):
            pltpu.sync_copy(
                zrow_v,
                acc_s.at[pl.ds(sub * rows_pt + j * (rows_pt // 8), rows_pt // 8)])
        tbase = wid * per_tile
        cbase = wid * n_chunks
        pltpu.sync_copy(row0g2_h.at[pl.ds(cbase, n_chunks)], ig_all)
        pltpu.sync_copy(row0c2_h.at[pl.ds(cbase, n_chunks)], ic_all)
        pltpu.sync_copy(colc2_h.at[pl.ds(cbase, n_chunks)], icc_all)
        pltpu.sync_copy(ncx_h, cx_v)
        pltpu.sync_copy(ncy_h, cy_v)
        pltpu.sync_copy(ncz_h, cz_v)
        plsc.subcore_barrier()

        c0 = jnp.zeros((L,), I32)
        c1 = jnp.full((L,), 1, I32)
        c2 = jnp.full((L,), 2, I32)

        def chunk(i, _):
            base = tbase + i * C
            for kk in range(C // L):
                idx = ig_all[i, pl.ds(kk * L, L)]
                cx = plsc.load_gather(cx_v, [idx])
                cy = plsc.load_gather(cy_v, [idx])
                cz = plsc.load_gather(cz_v, [idx])
                rows = lanes + kk * L
                plsc.store_scatter(crow_v, [rows, c0], cx)
                plsc.store_scatter(crow_v, [rows, c1], cy)
                plsc.store_scatter(crow_v, [rows, c2], cz)
                rho_v[pl.ds(kk * L, L)] = cx * cx + cy * cy + cz * cz
            pltpu.sync_copy(crow_v, acc_s.at[icc_all.at[i]], add=True)
            pltpu.sync_copy(cnt_v, acc_s.at[ic_all.at[i]], add=True)
            pltpu.sync_copy(rho_v, radial_h.at[pl.ds(base, C)])
            return 0
        lax.fori_loop(0, n_chunks, chunk, 0)

        plsc.subcore_barrier()
        pltpu.sync_copy(acc_s.at[pl.ds(sub * rows_pt, rows_pt)],
                        acc_out_h.at[core, pl.ds(sub * rows_pt, rows_pt)])

    return k


def _make_gather(NT, D, n_edges):
    per_tile = n_edges // NW
    C = 128
    n_chunks = per_tile // C
    mesh = plsc.VectorSubcoreMesh(core_axis_name="c", subcore_axis_name="s")

    @functools.partial(
        pl.kernel,
        out_type=jax.ShapeDtypeStruct((n_edges, D), F32),
        mesh=mesh,
        compiler_params=pltpu.CompilerParams(needs_layout_passes=False, use_tc_tiling_on_sc=False),
        scratch_types=[
            pltpu.VMEM((n_chunks, C), I32), pltpu.VMEM((n_chunks, C), I32),
            pltpu.VMEM((C, D), F32), pltpu.VMEM((C, D), F32),
            pltpu.VMEM((C, D), F32), pltpu.VMEM((C, D), F32),
            pltpu.SemaphoreType.DMA, pltpu.SemaphoreType.DMA,
            pltpu.SemaphoreType.DMA, pltpu.SemaphoreType.DMA,
            pltpu.SemaphoreType.DMA, pltpu.SemaphoreType.DMA,
        ],
    )
    def k(hA_h, hB_h, erg2_h, ecg2_h, pre_h,
          ier_all, iec_all, bufA0, bufB0, bufA1, bufB1,
          semA0, semB0, semA1, semB1, semO0, semO1):
        core = lax.axis_index("c")
        sub = lax.axis_index("s")
        wid = core * NS + sub
        tbase = wid * per_tile
        bufA = [bufA0, bufA1]
        bufB = [bufB0, bufB1]
        semA = [semA0, semA1]
        semB = [semB0, semB1]
        semO = [semO0, semO1]

        pltpu.sync_copy(erg2_h.at[pl.ds(wid * n_chunks, n_chunks)], ier_all)
        pltpu.sync_copy(ecg2_h.at[pl.ds(wid * n_chunks, n_chunks)], iec_all)

        def start(cur, slot):
            pltpu.async_copy(hA_h.at[ier_all.at[cur]], bufA[slot], semA[slot])
            pltpu.async_copy(hB_h.at[iec_all.at[cur]], bufB[slot], semB[slot])

        def drain_in(cur, slot):
            pltpu.make_async_copy(hA_h.at[ier_all.at[cur]], bufA[slot],
                                  semA[slot]).wait()
            pltpu.make_async_copy(hB_h.at[iec_all.at[cur]], bufB[slot],
                                  semB[slot]).wait()

        def drain_out(slot):
            pltpu.make_async_copy(bufA[slot], pre_h.at[pl.ds(tbase, C)],
                                  semO[slot]).wait()

        start(0, 0)

        def body(i, _):
            g = i * 2
            for b in range(2):
                cur = g + b
                nxt = cur + 1
                nslot = (b + 1) % 2

                @pl.when(cur < n_chunks)
                def _():
                    @pl.when(nxt < n_chunks)
                    def _():
                        @pl.when(nxt >= 2)
                        def _():
                            drain_out(nslot)
                        start(nxt, nslot)

                    drain_in(cur, b)

                    def radd(r, _):
                        for cc in range(D // L):
                            sl = pl.ds(cc * L, L)
                            bufA[b][r, sl] = bufA[b][r, sl] + bufB[b][r, sl]
                        return 0
                    lax.fori_loop(0, C, radd, 0)
                    pltpu.async_copy(bufA[b],
                                     pre_h.at[pl.ds(tbase + cur * C, C)],
                                     semO[b])
            return 0
        lax.fori_loop(0, (n_chunks + 1) // 2, body, 0)
        drain_out(0)
        drain_out(1)

    return k


def _make_agg1(ACC_R, D, ne):
    H = D // 2
    C = 128
    per_tile = ne // NS
    n_chunks = per_tile // C
    rows_pt = ACC_R // NS
    ZR = rows_pt // 8
    mesh = plsc.VectorSubcoreMesh(core_axis_name="c", subcore_axis_name="s")

    @functools.partial(
        pl.kernel,
        out_type=[jax.ShapeDtypeStruct((ACC_R, H), F32),
                  jax.ShapeDtypeStruct((ACC_R, H), F32)],
        mesh=mesh,
        compiler_params=pltpu.CompilerParams(needs_layout_passes=False, use_tc_tiling_on_sc=False),
        scratch_types=[
            pltpu.VMEM((C, H), F32), pltpu.VMEM((C, H), F32),
            pltpu.VMEM((n_chunks, C), I32),
            pltpu.VMEM((ZR, H), F32),
            pltpu.VMEM_SHARED((ACC_R, H), F32),
            pltpu.SemaphoreType.DMA, pltpu.SemaphoreType.DMA,
        ],
    )
    def k(eflo_h, efhi_h, ers_h, agglo_h, agghi_h,
          efb0, efb1, iea_all, zb_v, agg_s, semI0, semI1):
        core = lax.axis_index("c")
        sub = lax.axis_index("s")
        zero_l = jnp.zeros((L,), F32)
        efb = [efb0, efb1]
        semI = [semI0, semI1]

        def zfill(i, _):
            for cc in range(H // L):
                zb_v[i, pl.ds(cc * L, L)] = zero_l
            return 0
        lax.fori_loop(0, ZR, zfill, 0)
        for j in range(8):
            pltpu.sync_copy(zb_v, agg_s.at[pl.ds(sub * rows_pt + j * ZR, ZR)])
        pltpu.sync_copy(ers_h.at[pl.ds(sub * n_chunks, n_chunks)], iea_all)
        plsc.subcore_barrier()

        def agg_loop(ef_h):
            tbase = sub * per_tile

            def start(cur, slot):
                pltpu.async_copy(ef_h.at[pl.ds(tbase + cur * C, C)],
                                 efb[slot], semI[slot])

            def drain_in(slot):
                pltpu.make_async_copy(ef_h.at[pl.ds(tbase, C)], efb[slot],
                                      semI[slot]).wait()

            start(0, 0)

            def body(i, _):
                g = i * 2
                for b in range(2):
                    cur = g + b
                    nxt = cur + 1
                    nslot = (b + 1) % 2

                    @pl.when(cur < n_chunks)
                    def _():
                        @pl.when(nxt < n_chunks)
                        def _():
                            start(nxt, nslot)
                        drain_in(b)
                        pltpu.sync_copy(efb[b], agg_s.at[iea_all.at[cur]],
                                        add=True)
                return 0
            lax.fori_loop(0, (n_chunks + 1) // 2, body, 0)

        @pl.when(core == 0)
        def _():
            agg_loop(eflo_h)

        @pl.when(core == 1)
        def _():
            agg_loop(efhi_h)

        plsc.subcore_barrier()
        rsl = pl.ds(sub * rows_pt, rows_pt)

        @pl.when(core == 0)
        def _():
            pltpu.sync_copy(agg_s.at[rsl], agglo_h.at[rsl])

        @pl.when(core == 1)
        def _():
            pltpu.sync_copy(agg_s.at[rsl], agghi_h.at[rsl])

    return k


def _make_trans1(N, ACC_R, E_pad, sign):
    per_tile = E_pad // NW
    C = 128
    n_chunks = per_tile // C
    rows_pt = ACC_R // NS
    ZR = rows_pt // 8
    mesh = plsc.VectorSubcoreMesh(core_axis_name="c", subcore_axis_name="s")

    @functools.partial(
        pl.kernel,
        out_type=jax.ShapeDtypeStruct((NC, ACC_R, 16), F32),
        mesh=mesh,
        compiler_params=pltpu.CompilerParams(needs_layout_passes=False, use_tc_tiling_on_sc=False),
        scratch_types=[
            pltpu.VMEM((N,), F32), pltpu.VMEM((N,), F32), pltpu.VMEM((N,), F32),
            pltpu.VMEM((per_tile,), F32),
            pltpu.VMEM((n_chunks, C), I32), pltpu.VMEM((n_chunks, C), I32),
            pltpu.VMEM((C, 16), F32),
            pltpu.VMEM((ZR, 16), F32),
            pltpu.VMEM_SHARED((ACC_R, 16), F32),
        ],
    )
    def k(s_h, row0g2_h, key2_h, ncx_h, ncy_h, ncz_h, trans_h,
          cx_v, cy_v, cz_v, sv, ig_all, ik_all, rowA, zb_v, trans_s):
        core = lax.axis_index("c")
        sub = lax.axis_index("s")
        wid = core * NS + sub
        lanes = lax.iota(I32, L)
        zero_l = jnp.zeros((L,), F32)

        def zfill(i, _):
            zb_v[i] = zero_l
            return 0
        lax.fori_loop(0, ZR, zfill, 0)

        def zrow(i, _):
            rowA[i] = zero_l
            return 0
        lax.fori_loop(0, C, zrow, 0)

        for j in range(8):
            pltpu.sync_copy(zb_v, trans_s.at[pl.ds(sub * rows_pt + j * ZR, ZR)])
        tstart = wid * per_tile
        pltpu.sync_copy(s_h.at[pl.ds(tstart, per_tile)], sv)
        cbase = wid * n_chunks
        pltpu.sync_copy(row0g2_h.at[pl.ds(cbase, n_chunks)], ig_all)
        pltpu.sync_copy(key2_h.at[pl.ds(cbase, n_chunks)], ik_all)
        pltpu.sync_copy(ncx_h, cx_v)
        pltpu.sync_copy(ncy_h, cy_v)
        pltpu.sync_copy(ncz_h, cz_v)
        plsc.subcore_barrier()

        c0i = jnp.zeros((L,), I32)
        c1i = jnp.full((L,), 1, I32)
        c2i = jnp.full((L,), 2, I32)

        def tchunk(i, _):
            for kk in range(C // L):
                idx = ig_all[i, pl.ds(kk * L, L)]
                cx = plsc.load_gather(cx_v, [idx])
                cy = plsc.load_gather(cy_v, [idx])
                cz = plsc.load_gather(cz_v, [idx])
                a = sv[pl.ds(i * C + kk * L, L)] * sign
                rows = lanes + kk * L
                plsc.store_scatter(rowA, [rows, c0i], cx * a)
                plsc.store_scatter(rowA, [rows, c1i], cy * a)
                plsc.store_scatter(rowA, [rows, c2i], cz * a)
            pltpu.sync_copy(rowA, trans_s.at[ik_all.at[i]], add=True)
            return 0
        lax.fori_loop(0, n_chunks, tchunk, 0)

        plsc.subcore_barrier()
        rsl = pl.ds(sub * rows_pt, rows_pt)
        pltpu.sync_copy(trans_s.at[rsl], trans_h.at[core, rsl])

    return k


def kernel(x, hyperedge_feature, hyperedge_index, node_coord, Wlin, blin,
           We1, be1, We2, be2, Wc1, bc1, Wc2, Wn1, bn1, Wn2, bn2,
           centers, gamma, Wr, br):
    N, D = x.shape
    M = hyperedge_feature.shape[0]
    NT = N + M
    E = hyperedge_index.shape[1]
    ACC_DUMMY = NT
    ACC_R = ((NT + 1 + NS * 8 - 1) // (NS * 8)) * (NS * 8)
    E_pad = ((E + NW * 128 - 1) // (NW * 128)) * (NW * 128)
    TE_pad = ((2 * E + NW * 128 - 1) // (NW * 128)) * (NW * 128)

    row0 = hyperedge_index[0].astype(I32)
    col0 = hyperedge_index[1].astype(I32)
    ncx = node_coord[:, 0]
    ncy = node_coord[:, 1]
    ncz = node_coord[:, 2]

    row0g2 = _pad1(row0, E_pad, 0).reshape(-1, 128)
    row0c2 = _pad1(row0, E_pad, ACC_DUMMY).reshape(-1, 128)
    colc2 = _pad1(col0 + N, E_pad, ACC_DUMMY).reshape(-1, 128)
    er = jnp.concatenate([row0, col0 + N])
    ec = jnp.concatenate([col0 + N, row0])
    erg2 = _pad1(er, TE_pad, 0).reshape(-1, 128)
    ecg2 = _pad1(ec, TE_pad, 0).reshape(-1, 128)
    ers2 = _pad1(er, TE_pad, ACC_DUMMY).reshape(-1, 128)

    prep = _make_prep(N, M, D, 2000)
    h, hA, hB = prep(x, hyperedge_feature, Wlin, blin.reshape(1, D),
                     We1[:D], We1[D:2 * D], We1[2 * D:],
                     be1.reshape(1, D), br.reshape(1, D))

    stage0 = _make_stage0(N, NT, ACC_R, E_pad)
    accp, radial_p = stage0(ncx, ncy, ncz, row0g2, row0c2, colc2)

    aux = _make_aux(NT, 2000)
    mean3, cntc = aux(accp[0, :NT], accp[1, :NT])

    r1 = radial_p[:E]
    radial_dir = _pad1(jnp.concatenate([r1, r1]), TE_pad, 0.0).reshape(TE_pad, 1)

    SLAB0 = 40 * NW * 128
    SLAB1 = TE_pad - SLAB0
    CR0 = SLAB0 // 128
    gk0 = _make_gather(NT, D, SLAB0)
    gk1 = _make_gather(NT, D, SLAB1)
    pre0 = gk0(hA, hB, erg2[:CR0], ecg2[:CR0])
    pre1 = gk1(hA, hB, erg2[CR0:], ecg2[CR0:])

    wargs = (Wr, We1[2 * D:], We2, be2.reshape(1, D), Wc1, bc1.reshape(1, D),
             Wc2.reshape(1, D), gamma.reshape(1, 1), centers.reshape(1, 2))
    eflo0, efhi0, s0 = _make_mlp(SLAB0, D, 1024)(pre0, radial_dir[:SLAB0],
                                                 *wargs)
    eflo1, efhi1, s1 = _make_mlp(SLAB1, D, 1024)(pre1, radial_dir[SLAB0:],
                                                 *wargs)

    sa = _pad1(s0[:E, 0], E_pad, 0.0)
    sb = _pad1(jnp.concatenate([s0[E:, 0], s1[:, 0]])[: E], E_pad, 0.0)

    agglo0, agghi0 = _make_agg1(ACC_R, D, SLAB0)(eflo0, efhi0, ers2[:CR0])
    transA = _make_trans1(N, ACC_R, E_pad, 1.0)(sa, row0g2, row0g2,
                                                ncx, ncy, ncz)
    agglo1, agghi1 = _make_agg1(ACC_R, D, SLAB1)(eflo1, efhi1, ers2[CR0:])
    transB = _make_trans1(N, ACC_R, E_pad, -1.0)(sb, row0g2, colc2,
                                                 ncx, ncy, ncz)

    base_pad = jnp.concatenate([node_coord, jnp.zeros((M, 3), F32)], axis=0)
    nodek = _make_node(NT, D, 2000)
    hn, coord_top = nodek(h, agglo0[:NT], agglo1[:NT], agghi0[:NT],
                          agghi1[:NT], transA[0, :NT], transA[1, :NT],
                          transB[0, :NT], transB[1, :NT], cntc, base_pad,
                          Wn1[:D], Wn1[D:], bn1.reshape(1, D), Wn2,
                          bn2.reshape(1, D))

    coord = jnp.concatenate([coord_top, mean3[N:]], axis=0)
    return hn[:N], hn[N:], coord


# speedup vs baseline: 1.1195x; 1.1195x over previous
import functools

import jax
import jax.numpy as jnp
from jax import lax
from jax.experimental import pallas as pl
from jax.experimental.pallas import tpu as pltpu
from jax.experimental.pallas import tpu_sc as plsc

F32 = jnp.float32
BF16 = jnp.bfloat16
I32 = jnp.int32
EPS = 1e-8

NC, NS, L = 2, 16, 16
NW = NC * NS


def _pad1(a, n, v):
    return jnp.concatenate([a, jnp.full((n - a.shape[0],), v, a.dtype)])


def _make_prep(N, M, D, R):
    NT = N + M
    nb = NT // R
    nbx = N // R

    def body(x_ref, hef_ref, Wlin_ref, blin_ref, A_ref, B_ref, We1r_ref,
             be1_ref, br_ref, h_ref, hA_ref, hB_ref):
        pid = pl.program_id(0)
        is_node = pid < nbx
        xb = x_ref[:]
        hb = hef_ref[:]
        xw = jnp.dot(xb, Wlin_ref[:], preferred_element_type=F32) + blin_ref[:]
        h = jnp.where(is_node, xw, hb)
        h_ref[:] = h
        bias_e = be1_ref[:] + jnp.dot(br_ref[:], We1r_ref[:],
                                      preferred_element_type=F32)
        hA_ref[:] = jnp.dot(h, A_ref[:], preferred_element_type=F32) + bias_e
        hB_ref[:] = jnp.dot(h, B_ref[:], preferred_element_type=F32)

    row_spec = pl.BlockSpec((R, D), lambda i: (i, 0))
    x_spec = pl.BlockSpec((R, D), lambda i: (jnp.minimum(i, nbx - 1), 0))
    hef_spec = pl.BlockSpec((R, D), lambda i: (jnp.maximum(i - nbx, 0), 0))
    w_spec = pl.BlockSpec((D, D), lambda i: (0, 0))
    v_spec = pl.BlockSpec((1, D), lambda i: (0, 0))
    return pl.pallas_call(
        body,
        grid=(nb,),
        in_specs=[x_spec, hef_spec, w_spec, v_spec, w_spec, w_spec, w_spec,
                  v_spec, v_spec],
        out_specs=[row_spec, row_spec, row_spec],
        out_shape=[jax.ShapeDtypeStruct((NT, D), F32)] * 3,
    )


def _make_mlp(TE, D, BE):
    nb = TE // BE

    def body(pre_ref, rad_ref, Wr_ref, We1r_ref, We2_ref, be2_ref, Wc1_ref,
             bc1_ref, Wc2t_ref, gam_ref, cen_ref, eflo_ref, efhi_ref, s_ref):
        r = rad_ref[:]
        g = gam_ref[0, 0]
        c0 = cen_ref[0, 0]
        c1 = cen_ref[0, 1]
        Wrc = jnp.dot(Wr_ref[:], We1r_ref[:], preferred_element_type=F32)
        rbf0 = jnp.exp(-g * (r - c0) ** 2)
        rbf1 = jnp.exp(-g * (r - c1) ** 2)
        term = rbf0 * Wrc[0:1, :] + rbf1 * Wrc[1:2, :]
        e = pre_ref[:] + term
        e = e * jax.nn.sigmoid(e)
        ef = jnp.dot(e, We2_ref[:], preferred_element_type=F32) + be2_ref[:]
        ef = ef * jax.nn.sigmoid(ef)
        cm = jnp.dot(ef, Wc1_ref[:], preferred_element_type=F32) + bc1_ref[:]
        cm = cm * jax.nn.sigmoid(cm)
        cmt = jnp.tanh(jnp.sum(cm * Wc2t_ref[:], axis=1, keepdims=True))
        s_ref[:] = cmt / (jnp.sqrt(r) + EPS)
        eflo_ref[:] = ef[:, : D // 2]
        efhi_ref[:] = ef[:, D // 2:]

    row_spec = pl.BlockSpec((BE, D), lambda i: (i, 0))
    one_spec = pl.BlockSpec((BE, 1), lambda i: (i, 0))
    w_spec = pl.BlockSpec((D, D), lambda i: (0, 0))
    v_spec = pl.BlockSpec((1, D), lambda i: (0, 0))
    return pl.pallas_call(
        body,
        grid=(nb,),
        in_specs=[pl.BlockSpec((BE, D), lambda i: (i, 0)), one_spec,
                  pl.BlockSpec((2, D), lambda i: (0, 0)),
                  w_spec, w_spec, v_spec, w_spec, v_spec, v_spec,
                  pl.BlockSpec((1, 1), lambda i: (0, 0)),
                  pl.BlockSpec((1, 2), lambda i: (0, 0))],
        out_specs=[pl.BlockSpec((BE, D // 2), lambda i: (i, 0)),
                   pl.BlockSpec((BE, D // 2), lambda i: (i, 0)),
                   one_spec],
        out_shape=[jax.ShapeDtypeStruct((TE, D // 2), F32),
                   jax.ShapeDtypeStruct((TE, D // 2), F32),
                   jax.ShapeDtypeStruct((TE, 1), F32)],
    )


def _make_aux(NT, R):
    nb = NT // R

    def body(a0_ref, a1_ref, mean3_ref, cnt_ref):
        acc = a0_ref[:] + a1_ref[:]
        cnt = jnp.maximum(acc[:, 3:4], 1.0)
        mean3_ref[:] = acc[:, 0:3] / cnt
        cnt_ref[:] = cnt

    return pl.pallas_call(
        body,
        grid=(nb,),
        in_specs=[pl.BlockSpec((R, 16), lambda i: (i, 0))] * 2,
        out_specs=[pl.BlockSpec((R, 3), lambda i: (i, 0)),
                   pl.BlockSpec((R, 1), lambda i: (i, 0))],
        out_shape=[jax.ShapeDtypeStruct((NT, 3), F32),
                   jax.ShapeDtypeStruct((NT, 1), F32)],
    )


def _make_node(NT, D, R):
    nb = NT // R

    def body(h_ref, aglo_ref, aghi_ref, t0_ref, t1_ref, cnt_ref, base_ref,
             Wn1h_ref, Wn1a_ref, bn1_ref, Wn2_ref, bn2_ref,
             hn_ref, coord_ref):
        h = h_ref[:]
        agg = jnp.concatenate([aglo_ref[:], aghi_ref[:]], axis=1)
        mid = (jnp.dot(h, Wn1h_ref[:], preferred_element_type=F32)
               + jnp.dot(agg, Wn1a_ref[:], preferred_element_type=F32)
               + bn1_ref[:])
        mid = mid * jax.nn.sigmoid(mid)
        out = jnp.dot(mid, Wn2_ref[:], preferred_element_type=F32) + bn2_ref[:]
        hn_ref[:] = h + out
        tsum = t0_ref[:] + t1_ref[:]
        coord_ref[:] = base_ref[:] + tsum[:, 0:3] / cnt_ref[:]

    row_spec = pl.BlockSpec((R, D), lambda i: (i, 0))
    w_spec = pl.BlockSpec((D, D), lambda i: (0, 0))
    v_spec = pl.BlockSpec((1, D), lambda i: (0, 0))
    return pl.pallas_call(
        body,
        grid=(nb,),
        in_specs=[row_spec]
                 + [pl.BlockSpec((R, D // 2), lambda i: (i, 0))] * 2
                 + [pl.BlockSpec((R, 16), lambda i: (i, 0))] * 2
                 + [pl.BlockSpec((R, 1), lambda i: (i, 0)),
                    pl.BlockSpec((R, 3), lambda i: (i, 0)),
                    w_spec, w_spec, v_spec, w_spec, v_spec],
        out_specs=[row_spec, pl.BlockSpec((R, 3), lambda i: (i, 0))],
        out_shape=[jax.ShapeDtypeStruct((NT, D), F32),
                   jax.ShapeDtypeStruct((NT, 3), F32)],
    )


def _make_stage0(N, NT, ACC_R, E_pad):
    per_tile = E_pad // NW
    C = 128
    n_chunks = per_tile // C
    rows_pt = ACC_R // NS
    mesh = plsc.VectorSubcoreMesh(core_axis_name="c", subcore_axis_name="s")

    @functools.partial(
        pl.kernel,
        out_type=[jax.ShapeDtypeStruct((NC, ACC_R, 16), F32),
                  jax.ShapeDtypeStruct((E_pad,), F32)],
        mesh=mesh,
        compiler_params=pltpu.CompilerParams(needs_layout_passes=False, use_tc_tiling_on_sc=False),
        scratch_types=[
            pltpu.VMEM((N,), F32), pltpu.VMEM((N,), F32), pltpu.VMEM((N,), F32),
            pltpu.VMEM((n_chunks, C), I32), pltpu.VMEM((n_chunks, C), I32),
            pltpu.VMEM((n_chunks, C), I32),
            pltpu.VMEM((C, 16), F32), pltpu.VMEM((C, 16), F32),
            pltpu.VMEM((C,), F32),
            pltpu.VMEM((rows_pt // 8, 16), F32),
            pltpu.VMEM_SHARED((ACC_R, 16), F32),
        ],
    )
    def k(ncx_h, ncy_h, ncz_h, row0g2_h, row0c2_h, colc2_h,
          acc_out_h, radial_h,
          cx_v, cy_v, cz_v, ig_all, ic_all, icc_all, crow_v, cnt_v, rho_v,
          zrow_v, acc_s):
        core = lax.axis_index("c")
        sub = lax.axis_index("s")
        wid = core * NS + sub
        lanes = lax.iota(I32, L)
        one_l3 = jnp.where(lanes == 3, 1.0, 0.0).astype(F32)
        zero_l = jnp.zeros((L,), F32)

        def zfill(i, _):
            zrow_v[i] = zero_l
            return 0
        lax.fori_loop(0, rows_pt // 8, zfill, 0)

        def cfill(i, _):
            crow_v[i] = one_l3
            cnt_v[i] = one_l3
            return 0
        lax.fori_loop(0, C, cfill, 0)

        for j in range(8):
            pltpu.sync_copy(
                zrow_v,
                acc_s.at[pl.ds(sub * rows_pt + j * (rows_pt // 8), rows_pt // 8)])
        tbase = wid * per_tile
        cbase = wid * n_chunks
        pltpu.sync_copy(row0g2_h.at[pl.ds(cbase, n_chunks)], ig_all)
        pltpu.sync_copy(row0c2_h.at[pl.ds(cbase, n_chunks)], ic_all)
        pltpu.sync_copy(colc2_h.at[pl.ds(cbase, n_chunks)], icc_all)
        pltpu.sync_copy(ncx_h, cx_v)
        pltpu.sync_copy(ncy_h, cy_v)
        pltpu.sync_copy(ncz_h, cz_v)
        plsc.subcore_barrier()

        c0 = jnp.zeros((L,), I32)
        c1 = jnp.full((L,), 1, I32)
        c2 = jnp.full((L,), 2, I32)

        def chunk(i, _):
            base = tbase + i * C
            for kk in range(C // L):
                idx = ig_all[i, pl.ds(kk * L, L)]
                cx = plsc.load_gather(cx_v, [idx])
                cy = plsc.load_gather(cy_v, [idx])
                cz = plsc.load_gather(cz_v, [idx])
                rows = lanes + kk * L
                plsc.store_scatter(crow_v, [rows, c0], cx)
                plsc.store_scatter(crow_v, [rows, c1], cy)
                plsc.store_scatter(crow_v, [rows, c2], cz)
                rho_v[pl.ds(kk * L, L)] = cx * cx + cy * cy + cz * cz
            pltpu.sync_copy(crow_v, acc_s.at[icc_all.at[i]], add=True)
            pltpu.sync_copy(cnt_v, acc_s.at[ic_all.at[i]], add=True)
            pltpu.sync_copy(rho_v, radial_h.at[pl.ds(base, C)])
            return 0
        lax.fori_loop(0, n_chunks, chunk, 0)

        plsc.subcore_barrier()
        pltpu.sync_copy(acc_s.at[pl.ds(sub * rows_pt, rows_pt)],
                        acc_out_h.at[core, pl.ds(sub * rows_pt, rows_pt)])

    return k


def _make_gather(NT, D, n_edges):
    per_tile = n_edges // NW
    C = 128
    n_chunks = per_tile // C
    NB = 3
    mesh = plsc.VectorSubcoreMesh(core_axis_name="c", subcore_axis_name="s")

    @functools.partial(
        pl.kernel,
        out_type=jax.ShapeDtypeStruct((n_edges, D), F32),
        mesh=mesh,
        compiler_params=pltpu.CompilerParams(needs_layout_passes=False, use_tc_tiling_on_sc=False),
        scratch_types=[
            pltpu.VMEM((n_chunks, C), I32), pltpu.VMEM((n_chunks, C), I32),
            pltpu.VMEM((C, D), F32), pltpu.VMEM((C, D), F32),
            pltpu.VMEM((C, D), F32), pltpu.VMEM((C, D), F32),
            pltpu.VMEM((C, D), F32), pltpu.VMEM((C, D), F32),
            pltpu.SemaphoreType.DMA, pltpu.SemaphoreType.DMA,
            pltpu.SemaphoreType.DMA, pltpu.SemaphoreType.DMA,
            pltpu.SemaphoreType.DMA, pltpu.SemaphoreType.DMA,
            pltpu.SemaphoreType.DMA, pltpu.SemaphoreType.DMA,
            pltpu.SemaphoreType.DMA,
        ],
    )
    def k(hA_h, hB_h, erg2_h, ecg2_h, pre_h,
          ier_all, iec_all, bufA0, bufB0, bufA1, bufB1, bufA2, bufB2,
          semA0, semB0, semA1, semB1, semA2, semB2, semO0, semO1, semO2):
        core = lax.axis_index("c")
        sub = lax.axis_index("s")
        wid = core * NS + sub
        tbase = wid * per_tile
        bufA = [bufA0, bufA1, bufA2]
        bufB = [bufB0, bufB1, bufB2]
        semA = [semA0, semA1, semA2]
        semB = [semB0, semB1, semB2]
        semO = [semO0, semO1, semO2]

        pltpu.sync_copy(erg2_h.at[pl.ds(wid * n_chunks, n_chunks)], ier_all)
        pltpu.sync_copy(ecg2_h.at[pl.ds(wid * n_chunks, n_chunks)], iec_all)

        def start(cur, slot):
            pltpu.async_copy(hA_h.at[ier_all.at[cur]], bufA[slot], semA[slot])
            pltpu.async_copy(hB_h.at[iec_all.at[cur]], bufB[slot], semB[slot])

        def drain_in(cur, slot):
            pltpu.make_async_copy(hA_h.at[ier_all.at[cur]], bufA[slot],
                                  semA[slot]).wait()
            pltpu.make_async_copy(hB_h.at[iec_all.at[cur]], bufB[slot],
                                  semB[slot]).wait()

        def drain_out(slot):
            pltpu.make_async_copy(bufA[slot], pre_h.at[pl.ds(tbase, C)],
                                  semO[slot]).wait()

        start(0, 0)
        start(1, 1)

        def body(i, _):
            g = i * NB
            for b in range(NB):
                cur = g + b
                nxt = cur + 2
                nslot = (b + 2) % NB

                @pl.when(cur < n_chunks)
                def _():
                    @pl.when(nxt < n_chunks)
                    def _():
                        @pl.when(nxt >= NB)
                        def _():
                            drain_out(nslot)
                        start(nxt, nslot)

                    drain_in(cur, b)

                    def radd(r2, _):
                        for rr in range(2):
                            r = r2 * 2 + rr
                            for cc in range(D // L):
                                sl = pl.ds(cc * L, L)
                                bufA[b][r, sl] = bufA[b][r, sl] + bufB[b][r, sl]
                        return 0
                    lax.fori_loop(0, C // 2, radd, 0)
                    pltpu.async_copy(bufA[b],
                                     pre_h.at[pl.ds(tbase + cur * C, C)],
                                     semO[b])
            return 0
        lax.fori_loop(0, (n_chunks + NB - 1) // NB, body, 0)
        drain_out(0)
        drain_out(1)
        drain_out(2)

    return k


def _make_agg(ACC_R, D, ne0, ne1):
    H = D // 2
    C = 128
    nc0 = ne0 // (NS * C)
    nc1 = ne1 // (NS * C)
    n_chunks_total = nc0 + nc1
    rows_pt = ACC_R // NS
    ZR = rows_pt // 8
    mesh = plsc.VectorSubcoreMesh(core_axis_name="c", subcore_axis_name="s")

    @functools.partial(
        pl.kernel,
        out_type=[jax.ShapeDtypeStruct((ACC_R, H), F32),
                  jax.ShapeDtypeStruct((ACC_R, H), F32)],
        mesh=mesh,
        compiler_params=pltpu.CompilerParams(needs_layout_passes=False, use_tc_tiling_on_sc=False),
        scratch_types=[
            pltpu.VMEM((C, H), F32), pltpu.VMEM((C, H), F32),
            pltpu.VMEM((n_chunks_total, C), I32),
            pltpu.VMEM((ZR, H), F32),
            pltpu.VMEM_SHARED((ACC_R, H), F32),
            pltpu.SemaphoreType.DMA, pltpu.SemaphoreType.DMA,
        ],
    )
    def k(eflo0_h, eflo1_h, efhi0_h, efhi1_h, ers0_h, ers1_h,
          agglo_h, agghi_h,
          efb0, efb1, iea_all, zb_v, agg_s, semI0, semI1):
        core = lax.axis_index("c")
        sub = lax.axis_index("s")
        zero_l = jnp.zeros((L,), F32)
        efb = [efb0, efb1]
        semI = [semI0, semI1]

        def zfill(i, _):
            for cc in range(H // L):
                zb_v[i, pl.ds(cc * L, L)] = zero_l
            return 0
        lax.fori_loop(0, ZR, zfill, 0)
        for j in range(8):
            pltpu.sync_copy(zb_v, agg_s.at[pl.ds(sub * rows_pt + j * ZR, ZR)])
        pltpu.sync_copy(ers0_h.at[pl.ds(sub * nc0, nc0)],
                        iea_all.at[pl.ds(0, nc0)])
        pltpu.sync_copy(ers1_h.at[pl.ds(sub * nc1, nc1)],
                        iea_all.at[pl.ds(nc0, nc1)])
        plsc.subcore_barrier()

        def agg_loop(ef_h, n_chunks, per_tile, ioff):
            tbase = sub * per_tile

            def start(cur, slot):
                pltpu.async_copy(ef_h.at[pl.ds(tbase + cur * C, C)],
                                 efb[slot], semI[slot])

            def drain_in(slot):
                pltpu.make_async_copy(ef_h.at[pl.ds(tbase, C)], efb[slot],
                                      semI[slot]).wait()

            start(0, 0)

            def body(i, _):
                g = i * 2
                for b in range(2):
                    cur = g + b
                    nxt = cur + 1
                    nslot = (b + 1) % 2

                    @pl.when(cur < n_chunks)
                    def _():
                        @pl.when(nxt < n_chunks)
                        def _():
                            start(nxt, nslot)
                        drain_in(b)
                        pltpu.sync_copy(efb[b],
                                        agg_s.at[iea_all.at[ioff + cur]],
                                        add=True)
                return 0
            lax.fori_loop(0, (n_chunks + 1) // 2, body, 0)

        @pl.when(core == 0)
        def _():
            agg_loop(eflo0_h, nc0, ne0 // NS, 0)
            agg_loop(eflo1_h, nc1, ne1 // NS, nc0)

        @pl.when(core == 1)
        def _():
            agg_loop(efhi0_h, nc0, ne0 // NS, 0)
            agg_loop(efhi1_h, nc1, ne1 // NS, nc0)

        plsc.subcore_barrier()
        rsl = pl.ds(sub * rows_pt, rows_pt)

        @pl.when(core == 0)
        def _():
            pltpu.sync_copy(agg_s.at[rsl], agglo_h.at[rsl])

        @pl.when(core == 1)
        def _():
            pltpu.sync_copy(agg_s.at[rsl], agghi_h.at[rsl])

    return k


def _make_trans(N, ACC_R, E_pad):
    per_tile = (E_pad // 2) // NS
    C = 128
    n_chunks = per_tile // C
    rows_pt = ACC_R // NS
    ZR = rows_pt // 8
    mesh = plsc.VectorSubcoreMesh(core_axis_name="c", subcore_axis_name="s")

    @functools.partial(
        pl.kernel,
        out_type=jax.ShapeDtypeStruct((NC, ACC_R, 16), F32),
        mesh=mesh,
        compiler_params=pltpu.CompilerParams(needs_layout_passes=False, use_tc_tiling_on_sc=False),
        scratch_types=[
            pltpu.VMEM((N,), F32), pltpu.VMEM((N,), F32), pltpu.VMEM((N,), F32),
            pltpu.VMEM((per_tile,), F32), pltpu.VMEM((per_tile,), F32),
            pltpu.VMEM((n_chunks, C), I32), pltpu.VMEM((n_chunks, C), I32),
            pltpu.VMEM((C, 16), F32), pltpu.VMEM((C, 16), F32),
            pltpu.VMEM((ZR, 16), F32),
            pltpu.VMEM_SHARED((ACC_R, 16), F32),
        ],
    )
    def k(s1_h, s2_h, row0g2_h, colc2_h, ncx_h, ncy_h, ncz_h, trans_h,
          cx_v, cy_v, cz_v, sv1, sv2, ig_all, icc_all, rowA, rowB,
          zb_v, trans_s):
        core = lax.axis_index("c")
        sub = lax.axis_index("s")
        lanes = lax.iota(I32, L)
        zero_l = jnp.zeros((L,), F32)

        def zfill(i, _):
            zb_v[i] = zero_l
            return 0
        lax.fori_loop(0, ZR, zfill, 0)

        def zrow(i, _):
            rowA[i] = zero_l
            rowB[i] = zero_l
            return 0
        lax.fori_loop(0, C, zrow, 0)

        for j in range(8):
            pltpu.sync_copy(zb_v, trans_s.at[pl.ds(sub * rows_pt + j * ZR, ZR)])
        tstart = core * (E_pad // 2) + sub * per_tile
        pltpu.sync_copy(s1_h.at[pl.ds(tstart, per_tile)], sv1)
        pltpu.sync_copy(s2_h.at[pl.ds(tstart, per_tile)], sv2)
        cbase = tstart // C
        pltpu.sync_copy(row0g2_h.at[pl.ds(cbase, n_chunks)], ig_all)
        pltpu.sync_copy(colc2_h.at[pl.ds(cbase, n_chunks)], icc_all)
        pltpu.sync_copy(ncx_h, cx_v)
        pltpu.sync_copy(ncy_h, cy_v)
        pltpu.sync_copy(ncz_h, cz_v)
        plsc.subcore_barrier()

        c0i = jnp.zeros((L,), I32)
        c1i = jnp.full((L,), 1, I32)
        c2i = jnp.full((L,), 2, I32)

        def tchunk(i, _):
            for kk in range(C // L):
                idx = ig_all[i, pl.ds(kk * L, L)]
                cx = plsc.load_gather(cx_v, [idx])
                cy = plsc.load_gather(cy_v, [idx])
                cz = plsc.load_gather(cz_v, [idx])
                sl = pl.ds(i * C + kk * L, L)
                a = sv1[sl]
                nb = 0.0 - sv2[sl]
                rows = lanes + kk * L
                plsc.store_scatter(rowA, [rows, c0i], cx * a)
                plsc.store_scatter(rowA, [rows, c1i], cy * a)
                plsc.store_scatter(rowA, [rows, c2i], cz * a)
                plsc.store_scatter(rowB, [rows, c0i], cx * nb)
                plsc.store_scatter(rowB, [rows, c1i], cy * nb)
                plsc.store_scatter(rowB, [rows, c2i], cz * nb)
            pltpu.sync_copy(rowA, trans_s.at[ig_all.at[i]], add=True)
            pltpu.sync_copy(rowB, trans_s.at[icc_all.at[i]], add=True)
            return 0
        lax.fori_loop(0, n_chunks, tchunk, 0)

        plsc.subcore_barrier()
        rsl = pl.ds(sub * rows_pt, rows_pt)
        pltpu.sync_copy(trans_s.at[rsl], trans_h.at[core, rsl])

    return k


def kernel(x, hyperedge_feature, hyperedge_index, node_coord, Wlin, blin,
           We1, be1, We2, be2, Wc1, bc1, Wc2, Wn1, bn1, Wn2, bn2,
           centers, gamma, Wr, br):
    N, D = x.shape
    M = hyperedge_feature.shape[0]
    NT = N + M
    E = hyperedge_index.shape[1]
    ACC_DUMMY = NT
    ACC_R = ((NT + 1 + NS * 8 - 1) // (NS * 8)) * (NS * 8)
    E_pad = ((E + NW * 128 - 1) // (NW * 128)) * (NW * 128)
    TE_pad = ((2 * E + NW * 128 - 1) // (NW * 128)) * (NW * 128)

    row0 = hyperedge_index[0].astype(I32)
    col0 = hyperedge_index[1].astype(I32)
    ncx = node_coord[:, 0]
    ncy = node_coord[:, 1]
    ncz = node_coord[:, 2]

    row0g2 = _pad1(row0, E_pad, 0).reshape(-1, 128)
    row0c2 = _pad1(row0, E_pad, ACC_DUMMY).reshape(-1, 128)
    colc2 = _pad1(col0 + N, E_pad, ACC_DUMMY).reshape(-1, 128)
    er = jnp.concatenate([row0, col0 + N])
    ec = jnp.concatenate([col0 + N, row0])
    erg2 = _pad1(er, TE_pad, 0).reshape(-1, 128)
    ecg2 = _pad1(ec, TE_pad, 0).reshape(-1, 128)
    ers2 = _pad1(er, TE_pad, ACC_DUMMY).reshape(-1, 128)

    prep = _make_prep(N, M, D, 2000)
    h, hA, hB = prep(x, hyperedge_feature, Wlin, blin.reshape(1, D),
                     We1[:D], We1[D:2 * D], We1[2 * D:],
                     be1.reshape(1, D), br.reshape(1, D))

    stage0 = _make_stage0(N, NT, ACC_R, E_pad)
    accp, radial_p = stage0(ncx, ncy, ncz, row0g2, row0c2, colc2)

    aux = _make_aux(NT, 2000)
    mean3, cntc = aux(accp[0, :NT], accp[1, :NT])

    r1 = radial_p[:E]
    radial_dir = _pad1(jnp.concatenate([r1, r1]), TE_pad, 0.0).reshape(TE_pad, 1)

    SLAB0 = 40 * NW * 128
    SLAB1 = TE_pad - SLAB0
    CR0 = SLAB0 // 128
    gk0 = _make_gather(NT, D, SLAB0)
    gk1 = _make_gather(NT, D, SLAB1)
    pre0 = gk0(hA, hB, erg2[:CR0], ecg2[:CR0])
    pre1 = gk1(hA, hB, erg2[CR0:], ecg2[CR0:])

    wargs = (Wr, We1[2 * D:], We2, be2.reshape(1, D), Wc1, bc1.reshape(1, D),
             Wc2.reshape(1, D), gamma.reshape(1, 1), centers.reshape(1, 2))
    eflo0, efhi0, s0 = _make_mlp(SLAB0, D, 1024)(pre0, radial_dir[:SLAB0],
                                                 *wargs)
    eflo1, efhi1, s1 = _make_mlp(SLAB1, D, 1024)(pre1, radial_dir[SLAB0:],
                                                 *wargs)

    sa = _pad1(s0[:E, 0], E_pad, 0.0)
    sb = _pad1(jnp.concatenate([s0[E:, 0], s1[:, 0]])[: E], E_pad, 0.0)

    aggk = _make_agg(ACC_R, D, SLAB0, SLAB1)
    agglo, agghi = aggk(eflo0, eflo1, efhi0, efhi1, ers2[:CR0], ers2[CR0:])
    transk = _make_trans(N, ACC_R, E_pad)
    transp = transk(sa, sb, row0g2, colc2, ncx, ncy, ncz)

    base_pad = jnp.concatenate([node_coord, jnp.zeros((M, 3), F32)], axis=0)
    nodek = _make_node(NT, D, 2000)
    hn, coord_top = nodek(h, agglo[:NT], agghi[:NT], transp[0, :NT],
                          transp[1, :NT], cntc, base_pad,
                          Wn1[:D], Wn1[D:], bn1.reshape(1, D), Wn2,
                          bn2.reshape(1, D))

    coord = jnp.concatenate([coord_top, mean3[N:]], axis=0)
    return hn[:N], hn[N:], coord
